# baseline (speedup 1.0000x reference)
import functools

import jax
import jax.numpy as jnp
from jax import lax
from jax.experimental import pallas as pl
from jax.experimental.pallas import tpu as pltpu
from jax.experimental.pallas import tpu_sc as plsc

B = 4096
L = 200
H = 64
EPS = 1e-5
NC = 2
NS = 16
NW = NC * NS
ROWS_PER_W = B // NW
NG = (L + 15) // 16
LP = NG * 16


def _rsqrt(x):
    i = plsc.bitcast(x, jnp.int32)
    i = 0x5F3759DF - (i >> 1)
    y = plsc.bitcast(i, jnp.float32)
    for _ in range(3):
        y = y * (1.5 - 0.5 * x * y * y)
    return y


def kernel(input_ids, word_emb, pos_emb, gamma, beta):
    ids3 = input_ids.reshape(B, 2, L // 2).astype(jnp.int32)
    mesh = plsc.VectorSubcoreMesh(core_axis_name="c", subcore_axis_name="s")

    @functools.partial(
        pl.kernel,
        out_type=jax.ShapeDtypeStruct((B, L, H), jnp.float32),
        mesh=mesh,
        compiler_params=pltpu.CompilerParams(
            needs_layout_passes=False, use_tc_tiling_on_sc=False),
        scratch_types=[
            pltpu.VMEM((2, L // 2), jnp.int32),
            pltpu.VMEM((L, H), jnp.float32),
            pltpu.VMEM((L, H), jnp.float32),
            pltpu.VMEM((H, LP), jnp.float32),
            pltpu.VMEM((H, 16), jnp.float32),
            pltpu.VMEM((H,), jnp.float32),
            pltpu.VMEM((H,), jnp.float32),
            pltpu.VMEM((L, H), jnp.float32),
            pltpu.SemaphoreType.DMA,
        ],
    )
    def run(ids_hbm, wemb_hbm, pemb_hbm, gamma_hbm, beta_hbm, out_hbm,
            idx_v, rows_v, pos_v, pos_t, x_t, g_v, b_v, out_v, sem):
        wid = lax.axis_index("s") * NC + lax.axis_index("c")
        pltpu.sync_copy(pemb_hbm.at[pl.ds(0, L)], pos_v)
        pltpu.sync_copy(gamma_hbm, g_v)
        pltpu.sync_copy(beta_hbm, b_v)
        g_vec = [g_v[pl.ds(16 * i, 16)] for i in range(H // 16)]
        b_vec = [b_v[pl.ds(16 * i, 16)] for i in range(H // 16)]
        iota = lax.iota(jnp.int32, 16)

        def transpose_pos(g, carry):
            tok = jnp.minimum(g * 16 + iota, L - 1)
            for h in range(H):
                col = jnp.full((16,), h, jnp.int32)
                pos_t[h, pl.ds(g * 16, 16)] = plsc.load_gather(pos_v, [tok, col])
            return carry

        lax.fori_loop(0, NG, transpose_pos, 0)

        def row_body(r, carry):
            row = wid * ROWS_PER_W + r
            pltpu.sync_copy(ids_hbm.at[row], idx_v)
            cp0 = pltpu.async_copy(
                wemb_hbm.at[idx_v.at[0]], rows_v.at[pl.ds(0, L // 2)], sem)
            cp1 = pltpu.async_copy(
                wemb_hbm.at[idx_v.at[1]], rows_v.at[pl.ds(L // 2, L // 2)], sem)
            cp0.wait()
            cp1.wait()

            def grp_body(g, carry2):
                tok = jnp.minimum(g * 16 + iota, L - 1)
                s = jnp.zeros((16,), jnp.float32)
                ss = jnp.zeros((16,), jnp.float32)
                for h in range(H):
                    col = jnp.full((16,), h, jnp.int32)
                    w = plsc.load_gather(rows_v, [tok, col])
                    x = w + pos_t[h, pl.ds(g * 16, 16)]
                    x_t[h, pl.ds(0, 16)] = x
                    s = s + x
                    ss = ss + x * x
                mean = s * (1.0 / H)
                var = ss * (1.0 / H) - mean * mean
                inv = _rsqrt(var + EPS)
                for h in range(H):
                    col = jnp.full((16,), h, jnp.int32)
                    y = (x_t[h, pl.ds(0, 16)] - mean) * inv
                    y = y * g_vec[h // 16][h % 16] + b_vec[h // 16][h % 16]
                    plsc.store_scatter(out_v, [tok, col], y)
                return carry2

            lax.fori_loop(0, NG, grp_body, 0)
            pltpu.sync_copy(out_v, out_hbm.at[row])
            return carry

        lax.fori_loop(0, ROWS_PER_W, row_body, 0)

    return run(ids3, word_emb, pos_emb, gamma, beta)


# speedup vs baseline: 1.0794x; 1.0794x over previous
import functools

import jax
import jax.numpy as jnp
from jax import lax
from jax.experimental import pallas as pl
from jax.experimental.pallas import tpu as pltpu
from jax.experimental.pallas import tpu_sc as plsc

B = 4096
L = 200
H = 64
LH = L // 2
EPS = 1e-5
NC = 2
NS = 16
NW = NC * NS
ROWS_PER_W = B // NW
NG = (L + 15) // 16
LP = NG * 16


def _rsqrt(x):
    i = plsc.bitcast(x, jnp.int32)
    i = 0x5F3759DF - (i >> 1)
    y = plsc.bitcast(i, jnp.float32)
    for _ in range(3):
        y = y * (1.5 - 0.5 * x * y * y)
    return y


def kernel(input_ids, word_emb, pos_emb, gamma, beta):
    ids2 = input_ids.reshape(2 * B, LH).astype(jnp.int32)
    mesh = plsc.VectorSubcoreMesh(core_axis_name="c", subcore_axis_name="s")

    @functools.partial(
        pl.kernel,
        out_type=jax.ShapeDtypeStruct((B, L, H), jnp.float32),
        mesh=mesh,
        compiler_params=pltpu.CompilerParams(
            needs_layout_passes=False, use_tc_tiling_on_sc=False),
        scratch_types=[
            pltpu.VMEM((2 * ROWS_PER_W, LH), jnp.int32),
            pltpu.VMEM((2, L, H), jnp.float32),
            pltpu.VMEM((L, H), jnp.float32),
            pltpu.VMEM((H, LP), jnp.float32),
            pltpu.VMEM((H, 16), jnp.float32),
            pltpu.VMEM((H,), jnp.float32),
            pltpu.VMEM((H,), jnp.float32),
            pltpu.VMEM((2, L, H), jnp.float32),
            pltpu.SemaphoreType.DMA,
            pltpu.SemaphoreType.DMA,
            pltpu.SemaphoreType.DMA,
            pltpu.SemaphoreType.DMA,
        ],
    )
    def run(ids_hbm, wemb_hbm, pemb_hbm, gamma_hbm, beta_hbm, out_hbm,
            idx_v, rows_v, pos_v, pos_t, x_t, g_v, b_v, out_v,
            gsem0, gsem1, osem0, osem1):
        wid = lax.axis_index("s") * NC + lax.axis_index("c")
        row0 = wid * ROWS_PER_W
        pltpu.sync_copy(ids_hbm.at[pl.ds(2 * row0, 2 * ROWS_PER_W)], idx_v)
        pltpu.sync_copy(pemb_hbm.at[pl.ds(0, L)], pos_v)
        pltpu.sync_copy(gamma_hbm, g_v)
        pltpu.sync_copy(beta_hbm, b_v)
        g_vec = [g_v[pl.ds(16 * i, 16)] for i in range(H // 16)]
        b_vec = [b_v[pl.ds(16 * i, 16)] for i in range(H // 16)]
        iota = lax.iota(jnp.int32, 16)

        def transpose_pos(g, carry):
            tok = jnp.minimum(g * 16 + iota, L - 1)
            for h in range(H):
                col = jnp.full((16,), h, jnp.int32)
                pos_t[h, pl.ds(g * 16, 16)] = plsc.load_gather(pos_v, [tok, col])
            return carry

        lax.fori_loop(0, NG, transpose_pos, 0)

        def issue_gather(r, buf, sem):
            pltpu.async_copy(
                wemb_hbm.at[idx_v.at[2 * r]],
                rows_v.at[buf, pl.ds(0, LH)], sem)
            pltpu.async_copy(
                wemb_hbm.at[idx_v.at[2 * r + 1]],
                rows_v.at[buf, pl.ds(LH, LH)], sem)

        def wait_gather(r, buf, sem):
            for j in range(2):
                pltpu.make_async_copy(
                    wemb_hbm.at[idx_v.at[2 * r + j]],
                    rows_v.at[buf, pl.ds(j * LH, LH)], sem).wait()

        def compute_row(r, buf):
            def grp_body(g, carry):
                tok = jnp.minimum(g * 16 + iota, L - 1)
                s0 = jnp.zeros((16,), jnp.float32)
                s1 = jnp.zeros((16,), jnp.float32)
                q0 = jnp.zeros((16,), jnp.float32)
                q1 = jnp.zeros((16,), jnp.float32)
                for h in range(H):
                    col = jnp.full((16,), h, jnp.int32)
                    w = plsc.load_gather(rows_v.at[buf], [tok, col])
                    x = w + pos_t[h, pl.ds(g * 16, 16)]
                    x_t[h, pl.ds(0, 16)] = x
                    if h % 2 == 0:
                        s0 = s0 + x
                        q0 = q0 + x * x
                    else:
                        s1 = s1 + x
                        q1 = q1 + x * x
                mean = (s0 + s1) * (1.0 / H)
                var = (q0 + q1) * (1.0 / H) - mean * mean
                inv = _rsqrt(var + EPS)
                for h in range(H):
                    col = jnp.full((16,), h, jnp.int32)
                    y = (x_t[h, pl.ds(0, 16)] - mean) * inv
                    y = y * g_vec[h // 16][h % 16] + b_vec[h // 16][h % 16]
                    plsc.store_scatter(out_v.at[buf], [tok, col], y)
                return carry

            lax.fori_loop(0, NG, grp_body, 0)

        def wait_out(r, buf, sem):
            pltpu.make_async_copy(out_v.at[buf], out_hbm.at[row0 + r], sem).wait()

        def issue_out(r, buf, sem):
            pltpu.async_copy(out_v.at[buf], out_hbm.at[row0 + r], sem)

        issue_gather(0, 0, gsem0)

        def pair_body(i, carry):
            r = 2 * i
            issue_gather(r + 1, 1, gsem1)

            @pl.when(i > 0)
            def _():
                wait_out(r - 2, 0, osem0)
            wait_gather(r, 0, gsem0)
            compute_row(r, 0)
            issue_out(r, 0, osem0)

            @pl.when(i < ROWS_PER_W // 2 - 1)
            def _():
                issue_gather(r + 2, 0, gsem0)

            @pl.when(i > 0)
            def _():
                wait_out(r - 1, 1, osem1)
            wait_gather(r + 1, 1, gsem1)
            compute_row(r + 1, 1)
            issue_out(r + 1, 1, osem1)
            return carry

        lax.fori_loop(0, ROWS_PER_W // 2, pair_body, 0)
        wait_out(ROWS_PER_W - 2, 0, osem0)
        wait_out(ROWS_PER_W - 1, 1, osem1)

    return run(ids2, word_emb, pos_emb, gamma, beta)


# speedup vs baseline: 1.7028x; 1.5775x over previous
import functools

import jax
import jax.numpy as jnp
from jax import lax
from jax.experimental import pallas as pl
from jax.experimental.pallas import tpu as pltpu
from jax.experimental.pallas import tpu_sc as plsc

B = 4096
L = 200
H = 64
LH = L // 2
EPS = 1e-5
NC = 2
NS = 16
NW = NC * NS
ROWS_PER_W = B // NW
NG = (L + 15) // 16
LP = NG * 16


def _rsqrt(x):
    i = plsc.bitcast(x, jnp.int32)
    i = 0x5F3759DF - (i >> 1)
    y = plsc.bitcast(i, jnp.float32)
    for _ in range(3):
        y = y * (1.5 - 0.5 * x * y * y)
    return y


def kernel(input_ids, word_emb, pos_emb, gamma, beta):
    ids2 = input_ids.reshape(2 * B, LH).astype(jnp.int32)
    mesh = plsc.VectorSubcoreMesh(core_axis_name="c", subcore_axis_name="s")

    @functools.partial(
        pl.kernel,
        out_type=jax.ShapeDtypeStruct((B, L, H), jnp.float32),
        mesh=mesh,
        compiler_params=pltpu.CompilerParams(
            needs_layout_passes=False, use_tc_tiling_on_sc=False),
        scratch_types=[
            pltpu.VMEM((2 * ROWS_PER_W, LH), jnp.int32),
            pltpu.VMEM((2, L, H), jnp.float32),
            pltpu.VMEM((L, H), jnp.float32),
            pltpu.VMEM((H, LP), jnp.float32),
            pltpu.VMEM((H, 16), jnp.float32),
            pltpu.VMEM((H,), jnp.float32),
            pltpu.VMEM((H,), jnp.float32),
            pltpu.VMEM((H, 16), jnp.float32),
            pltpu.VMEM((H, 16), jnp.float32),
            pltpu.VMEM((2, L, H), jnp.float32),
            pltpu.SemaphoreType.DMA,
            pltpu.SemaphoreType.DMA,
            pltpu.SemaphoreType.DMA,
            pltpu.SemaphoreType.DMA,
        ],
    )
    def run(ids_hbm, wemb_hbm, pemb_hbm, gamma_hbm, beta_hbm, out_hbm,
            idx_v, rows_v, pos_v, pos_t, x_t, g_v, b_v, g_rot, b_rot, out_v,
            gsem0, gsem1, osem0, osem1):
        wid = lax.axis_index("s") * NC + lax.axis_index("c")
        row0 = wid * ROWS_PER_W
        pltpu.sync_copy(ids_hbm.at[pl.ds(2 * row0, 2 * ROWS_PER_W)], idx_v)
        pltpu.sync_copy(pemb_hbm.at[pl.ds(0, L)], pos_v)
        pltpu.sync_copy(gamma_hbm, g_v)
        pltpu.sync_copy(beta_hbm, b_v)
        iota = lax.iota(jnp.int32, 16)
        ecol = [(iota + h) % H for h in range(H)]

        for h in range(H):
            g_rot[h, pl.ds(0, 16)] = plsc.load_gather(g_v, [ecol[h]])
            b_rot[h, pl.ds(0, 16)] = plsc.load_gather(b_v, [ecol[h]])

        def transpose_pos(g, carry):
            tok = jnp.minimum(g * 16 + iota, L - 1)
            for h in range(H):
                pos_t[h, pl.ds(g * 16, 16)] = plsc.load_gather(
                    pos_v, [tok, ecol[h]])
            return carry

        lax.fori_loop(0, NG, transpose_pos, 0)

        def issue_gather(r, buf, sem):
            pltpu.async_copy(
                wemb_hbm.at[idx_v.at[2 * r]],
                rows_v.at[buf, pl.ds(0, LH)], sem)
            pltpu.async_copy(
                wemb_hbm.at[idx_v.at[2 * r + 1]],
                rows_v.at[buf, pl.ds(LH, LH)], sem)

        def wait_gather(r, buf, sem):
            for j in range(2):
                pltpu.make_async_copy(
                    wemb_hbm.at[idx_v.at[2 * r + j]],
                    rows_v.at[buf, pl.ds(j * LH, LH)], sem).wait()

        def compute_row(r, buf):
            def grp_body(g, carry):
                tok = jnp.minimum(g * 16 + iota, L - 1)
                s0 = jnp.zeros((16,), jnp.float32)
                s1 = jnp.zeros((16,), jnp.float32)
                q0 = jnp.zeros((16,), jnp.float32)
                q1 = jnp.zeros((16,), jnp.float32)
                for h in range(H):
                    w = plsc.load_gather(rows_v.at[buf], [tok, ecol[h]])
                    x = w + pos_t[h, pl.ds(g * 16, 16)]
                    x_t[h, pl.ds(0, 16)] = x
                    if h % 2 == 0:
                        s0 = s0 + x
                        q0 = q0 + x * x
                    else:
                        s1 = s1 + x
                        q1 = q1 + x * x
                mean = (s0 + s1) * (1.0 / H)
                var = (q0 + q1) * (1.0 / H) - mean * mean
                inv = _rsqrt(var + EPS)
                for h in range(H):
                    y = (x_t[h, pl.ds(0, 16)] - mean) * inv
                    y = y * g_rot[h, pl.ds(0, 16)] + b_rot[h, pl.ds(0, 16)]
                    plsc.store_scatter(out_v.at[buf], [tok, ecol[h]], y)
                return carry

            lax.fori_loop(0, NG, grp_body, 0)

        def wait_out(r, buf, sem):
            pltpu.make_async_copy(out_v.at[buf], out_hbm.at[row0 + r], sem).wait()

        def issue_out(r, buf, sem):
            pltpu.async_copy(out_v.at[buf], out_hbm.at[row0 + r], sem)

        issue_gather(0, 0, gsem0)

        def pair_body(i, carry):
            r = 2 * i
            issue_gather(r + 1, 1, gsem1)

            @pl.when(i > 0)
            def _():
                wait_out(r - 2, 0, osem0)
            wait_gather(r, 0, gsem0)
            compute_row(r, 0)
            issue_out(r, 0, osem0)

            @pl.when(i < ROWS_PER_W // 2 - 1)
            def _():
                issue_gather(r + 2, 0, gsem0)

            @pl.when(i > 0)
            def _():
                wait_out(r - 1, 1, osem1)
            wait_gather(r + 1, 1, gsem1)
            compute_row(r + 1, 1)
            issue_out(r + 1, 1, osem1)
            return carry

        lax.fori_loop(0, ROWS_PER_W // 2, pair_body, 0)
        wait_out(ROWS_PER_W - 2, 0, osem0)
        wait_out(ROWS_PER_W - 1, 1, osem1)

    return run(ids2, word_emb, pos_emb, gamma, beta)


# speedup vs baseline: 1.9827x; 1.1644x over previous
import functools

import jax
import jax.numpy as jnp
from jax import lax
from jax.experimental import pallas as pl
from jax.experimental.pallas import tpu as pltpu
from jax.experimental.pallas import tpu_sc as plsc

B = 4096
L = 200
H = 64
LH = L // 2
EPS = 1e-5
NC = 2
NS = 16
NW = NC * NS
ROWS_PER_W = B // NW
NG = (L + 15) // 16
LP = NG * 16
NBUF = 4


def _rsqrt(x):
    i = plsc.bitcast(x, jnp.int32)
    i = 0x5F3759DF - (i >> 1)
    y = plsc.bitcast(i, jnp.float32)
    for _ in range(3):
        y = y * (1.5 - 0.5 * x * y * y)
    return y


def kernel(input_ids, word_emb, pos_emb, gamma, beta):
    ids2 = input_ids.reshape(2 * B, LH).astype(jnp.int32)
    mesh = plsc.VectorSubcoreMesh(core_axis_name="c", subcore_axis_name="s")

    @functools.partial(
        pl.kernel,
        out_type=jax.ShapeDtypeStruct((B, L, H), jnp.float32),
        mesh=mesh,
        compiler_params=pltpu.CompilerParams(
            needs_layout_passes=False, use_tc_tiling_on_sc=False),
        scratch_types=[
            pltpu.VMEM((2 * ROWS_PER_W, LH), jnp.int32),
            pltpu.VMEM((NBUF, L, H), jnp.float32),
            pltpu.VMEM((H, LP), jnp.float32),
            pltpu.VMEM((H,), jnp.float32),
            pltpu.VMEM((H,), jnp.float32),
            pltpu.VMEM((H, 16), jnp.float32),
            pltpu.VMEM((H, 16), jnp.float32),
            pltpu.VMEM((2, L, H), jnp.float32),
            pltpu.SemaphoreType.DMA,
            pltpu.SemaphoreType.DMA,
            pltpu.SemaphoreType.DMA,
            pltpu.SemaphoreType.DMA,
            pltpu.SemaphoreType.DMA,
            pltpu.SemaphoreType.DMA,
        ],
    )
    def run(ids_hbm, wemb_hbm, pemb_hbm, gamma_hbm, beta_hbm, out_hbm,
            idx_v, rows_v, pos_t, g_v, b_v, g_rot, b_rot, out_v,
            gsem0, gsem1, gsem2, gsem3, osem0, osem1):
        gsems = [gsem0, gsem1, gsem2, gsem3]
        osems = [osem0, osem1]
        wid = lax.axis_index("s") * NC + lax.axis_index("c")
        row0 = wid * ROWS_PER_W
        pltpu.sync_copy(ids_hbm.at[pl.ds(2 * row0, 2 * ROWS_PER_W)], idx_v)
        pltpu.sync_copy(gamma_hbm, g_v)
        pltpu.sync_copy(beta_hbm, b_v)
        iota = lax.iota(jnp.int32, 16)
        ecol = [(iota + h) % H for h in range(H)]

        for h in range(H):
            g_rot[h, pl.ds(0, 16)] = plsc.load_gather(g_v, [ecol[h]])
            b_rot[h, pl.ds(0, 16)] = plsc.load_gather(b_v, [ecol[h]])

        pltpu.sync_copy(pemb_hbm.at[pl.ds(0, L)], rows_v.at[0])

        def transpose_pos(g, carry):
            tok = jnp.minimum(g * 16 + iota, L - 1)
            for h in range(H):
                pos_t[h, pl.ds(g * 16, 16)] = plsc.load_gather(
                    rows_v.at[0], [tok, ecol[h]])
            return carry

        lax.fori_loop(0, NG, transpose_pos, 0)

        def issue_gather(r, buf):
            pltpu.async_copy(
                wemb_hbm.at[idx_v.at[2 * r]],
                rows_v.at[buf, pl.ds(0, LH)], gsems[buf])
            pltpu.async_copy(
                wemb_hbm.at[idx_v.at[2 * r + 1]],
                rows_v.at[buf, pl.ds(LH, LH)], gsems[buf])

        def wait_gather(r, buf):
            for j in range(2):
                pltpu.make_async_copy(
                    wemb_hbm.at[idx_v.at[2 * r + j]],
                    rows_v.at[buf, pl.ds(j * LH, LH)], gsems[buf]).wait()

        def compute_row(r, buf, obuf):
            @plsc.parallel_loop(0, NG)
            def grp_body(g):
                tok = jnp.minimum(g * 16 + iota, L - 1)
                s0 = jnp.zeros((16,), jnp.float32)
                s1 = jnp.zeros((16,), jnp.float32)
                q0 = jnp.zeros((16,), jnp.float32)
                q1 = jnp.zeros((16,), jnp.float32)
                for h in range(H):
                    w = plsc.load_gather(rows_v.at[buf], [tok, ecol[h]])
                    x = w + pos_t[h, pl.ds(g * 16, 16)]
                    if h % 2 == 0:
                        s0 = s0 + x
                        q0 = q0 + x * x
                    else:
                        s1 = s1 + x
                        q1 = q1 + x * x
                mean = (s0 + s1) * (1.0 / H)
                var = (q0 + q1) * (1.0 / H) - mean * mean
                inv = _rsqrt(var + EPS)
                for h in range(H):
                    w = plsc.load_gather(rows_v.at[buf], [tok, ecol[h]])
                    x = w + pos_t[h, pl.ds(g * 16, 16)]
                    y = (x - mean) * inv
                    y = y * g_rot[h, pl.ds(0, 16)] + b_rot[h, pl.ds(0, 16)]
                    plsc.store_scatter(out_v.at[obuf], [tok, ecol[h]], y)

        def wait_out(r, obuf):
            pltpu.make_async_copy(
                out_v.at[obuf], out_hbm.at[row0 + r], osems[obuf]).wait()

        def issue_out(r, obuf):
            pltpu.async_copy(out_v.at[obuf], out_hbm.at[row0 + r], osems[obuf])

        for b in range(NBUF - 1):
            issue_gather(b, b)

        def quad_body(i, carry):
            for b in range(NBUF):
                r = NBUF * i + b
                nb = (b + NBUF - 1) % NBUF

                @pl.when(r + NBUF - 1 < ROWS_PER_W)
                def _():
                    issue_gather(r + NBUF - 1, nb)
                wait_gather(r, b)
                ob = b % 2

                @pl.when(r > 1)
                def _():
                    wait_out(r - 2, ob)
                compute_row(r, b, ob)
                issue_out(r, ob)
            return carry

        lax.fori_loop(0, ROWS_PER_W // NBUF, quad_body, 0)
        wait_out(ROWS_PER_W - 2, 0)
        wait_out(ROWS_PER_W - 1, 1)

    return run(ids2, word_emb, pos_emb, gamma, beta)


# speedup vs baseline: 1.9997x; 1.0086x over previous
import functools

import jax
import jax.numpy as jnp
from jax import lax
from jax.experimental import pallas as pl
from jax.experimental.pallas import tpu as pltpu
from jax.experimental.pallas import tpu_sc as plsc

B = 4096
L = 200
H = 64
LH = L // 2
EPS = 1e-5
NC = 2
NS = 16
NW = NC * NS
ROWS_PER_W = B // NW
NG = (L + 15) // 16
LP = NG * 16
NBUF = 4


def _rsqrt(x):
    i = plsc.bitcast(x, jnp.int32)
    i = 0x5F3759DF - (i >> 1)
    y = plsc.bitcast(i, jnp.float32)
    for _ in range(3):
        y = y * (1.5 - 0.5 * x * y * y)
    return y


def kernel(input_ids, word_emb, pos_emb, gamma, beta):
    ids2 = input_ids.reshape(2 * B, LH).astype(jnp.int32)
    mesh = plsc.VectorSubcoreMesh(core_axis_name="c", subcore_axis_name="s")

    @functools.partial(
        pl.kernel,
        out_type=jax.ShapeDtypeStruct((B, L, H), jnp.float32),
        mesh=mesh,
        compiler_params=pltpu.CompilerParams(
            needs_layout_passes=False, use_tc_tiling_on_sc=False),
        scratch_types=[
            pltpu.VMEM((2 * ROWS_PER_W, LH), jnp.int32),
            pltpu.VMEM((NBUF, L, H), jnp.float32),
            pltpu.VMEM((H, LP), jnp.float32),
            pltpu.VMEM((H,), jnp.float32),
            pltpu.VMEM((H,), jnp.float32),
            pltpu.VMEM((H, 16), jnp.float32),
            pltpu.VMEM((H, 16), jnp.float32),
            pltpu.VMEM((2, L, H), jnp.float32),
            pltpu.SemaphoreType.DMA,
            pltpu.SemaphoreType.DMA,
            pltpu.SemaphoreType.DMA,
            pltpu.SemaphoreType.DMA,
            pltpu.SemaphoreType.DMA,
            pltpu.SemaphoreType.DMA,
        ],
    )
    def run(ids_hbm, wemb_hbm, pemb_hbm, gamma_hbm, beta_hbm, out_hbm,
            idx_v, rows_v, pos_t, g_v, b_v, g_rot, b_rot, out_v,
            gsem0, gsem1, gsem2, gsem3, osem0, osem1):
        gsems = [gsem0, gsem1, gsem2, gsem3]
        osems = [osem0, osem1]
        wid = lax.axis_index("s") * NC + lax.axis_index("c")
        row0 = wid * ROWS_PER_W
        pltpu.sync_copy(ids_hbm.at[pl.ds(2 * row0, 2 * ROWS_PER_W)], idx_v)
        pltpu.sync_copy(gamma_hbm, g_v)
        pltpu.sync_copy(beta_hbm, b_v)
        iota = lax.iota(jnp.int32, 16)

        def ecol(h):
            return (iota + h) & (H - 1)

        for h in range(H):
            g_rot[h, pl.ds(0, 16)] = plsc.load_gather(g_v, [ecol(h)])
            b_rot[h, pl.ds(0, 16)] = plsc.load_gather(b_v, [ecol(h)])

        pltpu.sync_copy(pemb_hbm.at[pl.ds(0, L)], rows_v.at[0])

        def transpose_pos(g, carry):
            tok = jnp.minimum(g * 16 + iota, L - 1)
            for h in range(H):
                pos_t[h, pl.ds(g * 16, 16)] = plsc.load_gather(
                    rows_v.at[0], [tok, ecol(h)])
            return carry

        lax.fori_loop(0, NG, transpose_pos, 0)

        def issue_gather(r, buf):
            pltpu.async_copy(
                wemb_hbm.at[idx_v.at[2 * r]],
                rows_v.at[buf, pl.ds(0, LH)], gsems[buf])
            pltpu.async_copy(
                wemb_hbm.at[idx_v.at[2 * r + 1]],
                rows_v.at[buf, pl.ds(LH, LH)], gsems[buf])

        def wait_gather(r, buf):
            for j in range(2):
                pltpu.make_async_copy(
                    wemb_hbm.at[idx_v.at[2 * r + j]],
                    rows_v.at[buf, pl.ds(j * LH, LH)], gsems[buf]).wait()

        def compute_row(r, buf, obuf):
            @plsc.parallel_loop(0, NG)
            def grp_body(g):
                tok = jnp.minimum(g * 16 + iota, L - 1)
                s = [jnp.zeros((16,), jnp.float32) for _ in range(4)]
                q = [jnp.zeros((16,), jnp.float32) for _ in range(4)]
                for h in range(H):
                    w = plsc.load_gather(rows_v.at[buf], [tok, ecol(h)])
                    x = w + pos_t[h, pl.ds(g * 16, 16)]
                    s[h % 4] = s[h % 4] + x
                    q[h % 4] = q[h % 4] + x * x
                mean = ((s[0] + s[1]) + (s[2] + s[3])) * (1.0 / H)
                var = ((q[0] + q[1]) + (q[2] + q[3])) * (1.0 / H) - mean * mean
                inv = _rsqrt(var + EPS)
                for h in range(H):
                    w = plsc.load_gather(rows_v.at[buf], [tok, ecol(h)])
                    x = w + pos_t[h, pl.ds(g * 16, 16)]
                    y = (x - mean) * inv
                    y = y * g_rot[h, pl.ds(0, 16)] + b_rot[h, pl.ds(0, 16)]
                    plsc.store_scatter(out_v.at[obuf], [tok, ecol(h)], y)

        def wait_out(r, obuf):
            pltpu.make_async_copy(
                out_v.at[obuf], out_hbm.at[row0 + r], osems[obuf]).wait()

        def issue_out(r, obuf):
            pltpu.async_copy(out_v.at[obuf], out_hbm.at[row0 + r], osems[obuf])

        for b in range(NBUF - 1):
            issue_gather(b, b)

        def quad_body(i, carry):
            for b in range(NBUF):
                r = NBUF * i + b
                nb = (b + NBUF - 1) % NBUF

                @pl.when(r + NBUF - 1 < ROWS_PER_W)
                def _():
                    issue_gather(r + NBUF - 1, nb)
                wait_gather(r, b)
                ob = b % 2

                @pl.when(r > 1)
                def _():
                    wait_out(r - 2, ob)
                compute_row(r, b, ob)
                issue_out(r, ob)
            return carry

        lax.fori_loop(0, ROWS_PER_W // NBUF, quad_body, 0)
        wait_out(ROWS_PER_W - 2, 0)
        wait_out(ROWS_PER_W - 1, 1)

    return run(ids2, word_emb, pos_emb, gamma, beta)


# speedup vs baseline: 3.8403x; 1.9204x over previous
import functools

import jax
import jax.numpy as jnp
from jax import lax
from jax.experimental import pallas as pl
from jax.experimental.pallas import tpu as pltpu
from jax.experimental.pallas import tpu_sc as plsc

B = 4096
L = 200
H = 64
LH = L // 2
EPS = 1e-5
NC = 2
NS = 16
NW = NC * NS
ROWS_PER_W = B // NW
NG = (L + 15) // 16
LP = NG * 16
NBUF = 4


def _rsqrt(x):
    i = plsc.bitcast(x, jnp.int32)
    i = 0x5F3759DF - (i >> 1)
    y = plsc.bitcast(i, jnp.float32)
    for _ in range(3):
        y = y * (1.5 - 0.5 * x * y * y)
    return y


def kernel(input_ids, word_emb, pos_emb, gamma, beta):
    ids2 = input_ids.reshape(2 * B, LH).astype(jnp.int32)
    mesh = plsc.VectorSubcoreMesh(core_axis_name="c", subcore_axis_name="s")

    @functools.partial(
        pl.kernel,
        out_type=jax.ShapeDtypeStruct((B, L, H), jnp.float32),
        mesh=mesh,
        compiler_params=pltpu.CompilerParams(
            needs_layout_passes=False, use_tc_tiling_on_sc=False),
        scratch_types=[
            pltpu.VMEM((2 * ROWS_PER_W, LH), jnp.int32),
            pltpu.VMEM((NBUF, L, H), jnp.float32),
            pltpu.VMEM((H, LP), jnp.float32),
            pltpu.VMEM((H,), jnp.float32),
            pltpu.VMEM((H,), jnp.float32),
            pltpu.VMEM((H, 16), jnp.float32),
            pltpu.VMEM((H, 16), jnp.float32),
            pltpu.VMEM((2, L, H), jnp.float32),
            pltpu.SemaphoreType.DMA,
            pltpu.SemaphoreType.DMA,
            pltpu.SemaphoreType.DMA,
            pltpu.SemaphoreType.DMA,
            pltpu.SemaphoreType.DMA,
            pltpu.SemaphoreType.DMA,
        ],
    )
    def run(ids_hbm, wemb_hbm, pemb_hbm, gamma_hbm, beta_hbm, out_hbm,
            idx_v, rows_v, pos_t, g_v, b_v, g_rot, b_rot, out_v,
            gsem0, gsem1, gsem2, gsem3, osem0, osem1):
        gsems = [gsem0, gsem1, gsem2, gsem3]
        osems = [osem0, osem1]
        wid = lax.axis_index("s") * NC + lax.axis_index("c")
        row0 = wid * ROWS_PER_W
        pltpu.sync_copy(ids_hbm.at[pl.ds(2 * row0, 2 * ROWS_PER_W)], idx_v)
        pltpu.sync_copy(gamma_hbm, g_v)
        pltpu.sync_copy(beta_hbm, b_v)
        iota = lax.iota(jnp.int32, 16)

        def ecol(h):
            return (iota + h) & (H - 1)

        def build_rot(h, carry):
            g_rot[h, pl.ds(0, 16)] = plsc.load_gather(g_v, [ecol(h)])
            b_rot[h, pl.ds(0, 16)] = plsc.load_gather(b_v, [ecol(h)])
            return carry

        lax.fori_loop(0, H, build_rot, 0)

        pltpu.sync_copy(pemb_hbm.at[pl.ds(0, L)], rows_v.at[0])

        def transpose_pos(g, carry):
            tok = jnp.minimum(g * 16 + iota, L - 1)

            def tp_h(h, c):
                pos_t[h, pl.ds(g * 16, 16)] = plsc.load_gather(
                    rows_v.at[0], [tok, ecol(h)])
                return c

            lax.fori_loop(0, H, tp_h, 0)
            return carry

        lax.fori_loop(0, NG, transpose_pos, 0)

        def issue_gather(r, buf):
            pltpu.async_copy(
                wemb_hbm.at[idx_v.at[2 * r]],
                rows_v.at[buf, pl.ds(0, LH)], gsems[buf])
            pltpu.async_copy(
                wemb_hbm.at[idx_v.at[2 * r + 1]],
                rows_v.at[buf, pl.ds(LH, LH)], gsems[buf])

        def wait_gather(r, buf):
            for j in range(2):
                pltpu.make_async_copy(
                    wemb_hbm.at[idx_v.at[2 * r + j]],
                    rows_v.at[buf, pl.ds(j * LH, LH)], gsems[buf]).wait()

        def compute_row(r, buf, obuf):
            def grp_body(g, carry):
                tok = jnp.minimum(g * 16 + iota, L - 1)
                zero = jnp.zeros((16,), jnp.float32)

                @plsc.parallel_loop(0, H, unroll=8,
                                    carry=(zero, zero, zero, zero))
                def pass1(h, c):
                    s0, q0, s1, q1 = c
                    w = plsc.load_gather(rows_v.at[buf], [tok, ecol(h)])
                    x = w + pos_t[h, pl.ds(g * 16, 16)]
                    return (s1, q1, s0 + x, q0 + x * x)

                s0, q0, s1, q1 = pass1
                mean = (s0 + s1) * (1.0 / H)
                var = (q0 + q1) * (1.0 / H) - mean * mean
                inv = _rsqrt(var + EPS)

                @plsc.parallel_loop(0, H, unroll=8)
                def pass2(h):
                    w = plsc.load_gather(rows_v.at[buf], [tok, ecol(h)])
                    x = w + pos_t[h, pl.ds(g * 16, 16)]
                    y = (x - mean) * inv
                    y = y * g_rot[h, pl.ds(0, 16)] + b_rot[h, pl.ds(0, 16)]
                    plsc.store_scatter(out_v.at[obuf], [tok, ecol(h)], y)
                return carry

            lax.fori_loop(0, NG, grp_body, 0)

        def wait_out(r, obuf):
            pltpu.make_async_copy(
                out_v.at[obuf], out_hbm.at[row0 + r], osems[obuf]).wait()

        def issue_out(r, obuf):
            pltpu.async_copy(out_v.at[obuf], out_hbm.at[row0 + r], osems[obuf])

        for b in range(NBUF - 1):
            issue_gather(b, b)

        def quad_body(i, carry):
            for b in range(NBUF):
                r = NBUF * i + b
                nb = (b + NBUF - 1) % NBUF

                @pl.when(r + NBUF - 1 < ROWS_PER_W)
                def _():
                    issue_gather(r + NBUF - 1, nb)
                wait_gather(r, b)
                ob = b % 2

                @pl.when(r > 1)
                def _():
                    wait_out(r - 2, ob)
                compute_row(r, b, ob)
                issue_out(r, ob)
            return carry

        lax.fori_loop(0, ROWS_PER_W // NBUF, quad_body, 0)
        wait_out(ROWS_PER_W - 2, 0)
        wait_out(ROWS_PER_W - 1, 1)

    return run(ids2, word_emb, pos_emb, gamma, beta)


# speedup vs baseline: 3.9208x; 1.0210x over previous
import functools

import jax
import jax.numpy as jnp
from jax import lax
from jax.experimental import pallas as pl
from jax.experimental.pallas import tpu as pltpu
from jax.experimental.pallas import tpu_sc as plsc

B = 4096
L = 200
H = 64
LH = L // 2
EPS = 1e-5
NC = 2
NS = 16
NW = NC * NS
ROWS_PER_W = B // NW
NG = (L + 15) // 16
LP = NG * 16
NBUF = 4


def _rsqrt(x):
    i = plsc.bitcast(x, jnp.int32)
    i = 0x5F3759DF - (i >> 1)
    y = plsc.bitcast(i, jnp.float32)
    for _ in range(3):
        y = y * (1.5 - 0.5 * x * y * y)
    return y


def kernel(input_ids, word_emb, pos_emb, gamma, beta):
    ids2 = input_ids.reshape(2 * B, LH).astype(jnp.int32)
    mesh = plsc.VectorSubcoreMesh(core_axis_name="c", subcore_axis_name="s")

    @functools.partial(
        pl.kernel,
        out_type=jax.ShapeDtypeStruct((B, L, H), jnp.float32),
        mesh=mesh,
        compiler_params=pltpu.CompilerParams(
            needs_layout_passes=False, use_tc_tiling_on_sc=False),
        scratch_types=[
            pltpu.VMEM((2 * ROWS_PER_W, LH), jnp.int32),
            pltpu.VMEM((NBUF, L, H), jnp.float32),
            pltpu.VMEM((H, LP), jnp.float32),
            pltpu.VMEM((H,), jnp.float32),
            pltpu.VMEM((H,), jnp.float32),
            pltpu.VMEM((H, 16), jnp.float32),
            pltpu.VMEM((H, 16), jnp.float32),
            pltpu.VMEM((2, L, H), jnp.float32),
            pltpu.SemaphoreType.DMA,
            pltpu.SemaphoreType.DMA,
            pltpu.SemaphoreType.DMA,
            pltpu.SemaphoreType.DMA,
            pltpu.SemaphoreType.DMA,
            pltpu.SemaphoreType.DMA,
        ],
    )
    def run(ids_hbm, wemb_hbm, pemb_hbm, gamma_hbm, beta_hbm, out_hbm,
            idx_v, rows_v, pos_t, g_v, b_v, g_rot, b_rot, out_v,
            gsem0, gsem1, gsem2, gsem3, osem0, osem1):
        gsems = [gsem0, gsem1, gsem2, gsem3]
        osems = [osem0, osem1]
        wid = lax.axis_index("s") * NC + lax.axis_index("c")
        row0 = wid * ROWS_PER_W
        pltpu.sync_copy(ids_hbm.at[pl.ds(2 * row0, 2 * ROWS_PER_W)], idx_v)
        pltpu.sync_copy(gamma_hbm, g_v)
        pltpu.sync_copy(beta_hbm, b_v)
        iota = lax.iota(jnp.int32, 16)

        def ecol(h):
            return (iota + h) & (H - 1)

        def build_rot(h, carry):
            g_rot[h, pl.ds(0, 16)] = plsc.load_gather(g_v, [ecol(h)])
            b_rot[h, pl.ds(0, 16)] = plsc.load_gather(b_v, [ecol(h)])
            return carry

        lax.fori_loop(0, H, build_rot, 0)

        pltpu.sync_copy(pemb_hbm.at[pl.ds(0, L)], rows_v.at[0])

        def transpose_pos(g, carry):
            tok = jnp.minimum(g * 16 + iota, L - 1)

            def tp_h(h, c):
                pos_t[h, pl.ds(g * 16, 16)] = plsc.load_gather(
                    rows_v.at[0], [tok, ecol(h)])
                return c

            lax.fori_loop(0, H, tp_h, 0)
            return carry

        lax.fori_loop(0, NG, transpose_pos, 0)

        def issue_gather(r, buf):
            pltpu.async_copy(
                wemb_hbm.at[idx_v.at[2 * r]],
                rows_v.at[buf, pl.ds(0, LH)], gsems[buf])
            pltpu.async_copy(
                wemb_hbm.at[idx_v.at[2 * r + 1]],
                rows_v.at[buf, pl.ds(LH, LH)], gsems[buf])

        def wait_gather(r, buf):
            for j in range(2):
                pltpu.make_async_copy(
                    wemb_hbm.at[idx_v.at[2 * r + j]],
                    rows_v.at[buf, pl.ds(j * LH, LH)], gsems[buf]).wait()

        def compute_row(r, buf, obuf):
            def grp_body(g, carry):
                tok = jnp.minimum(g * 16 + iota, L - 1)
                zero = jnp.zeros((16,), jnp.float32)

                @plsc.parallel_loop(0, H, unroll=8,
                                    carry=(zero, zero, zero, zero))
                def pass1(h, c):
                    s0, q0, s1, q1 = c
                    w = plsc.load_gather(rows_v.at[buf], [tok, ecol(h)])
                    x = w + pos_t[h, pl.ds(g * 16, 16)]
                    return (s1, q1, s0 + x, q0 + x * x)

                s0, q0, s1, q1 = pass1
                mean = (s0 + s1) * (1.0 / H)
                var = (q0 + q1) * (1.0 / H) - mean * mean
                inv = _rsqrt(var + EPS)

                @plsc.parallel_loop(0, H, unroll=16)
                def pass2(h):
                    w = plsc.load_gather(rows_v.at[buf], [tok, ecol(h)])
                    x = w + pos_t[h, pl.ds(g * 16, 16)]
                    y = (x - mean) * inv
                    y = y * g_rot[h, pl.ds(0, 16)] + b_rot[h, pl.ds(0, 16)]
                    plsc.store_scatter(out_v.at[obuf], [tok, ecol(h)], y)
                return carry

            lax.fori_loop(0, NG, grp_body, 0)

        def wait_out(r, obuf):
            pltpu.make_async_copy(
                out_v.at[obuf], out_hbm.at[row0 + r], osems[obuf]).wait()

        def issue_out(r, obuf):
            pltpu.async_copy(out_v.at[obuf], out_hbm.at[row0 + r], osems[obuf])

        for b in range(NBUF - 1):
            issue_gather(b, b)

        def quad_body(i, carry):
            for b in range(NBUF):
                r = NBUF * i + b
                nb = (b + NBUF - 1) % NBUF

                @pl.when(r + NBUF - 1 < ROWS_PER_W)
                def _():
                    issue_gather(r + NBUF - 1, nb)
                wait_gather(r, b)
                ob = b % 2

                @pl.when(r > 1)
                def _():
                    wait_out(r - 2, ob)
                compute_row(r, b, ob)
                issue_out(r, ob)
            return carry

        lax.fori_loop(0, ROWS_PER_W // NBUF, quad_body, 0)
        wait_out(ROWS_PER_W - 2, 0)
        wait_out(ROWS_PER_W - 1, 1)

    return run(ids2, word_emb, pos_emb, gamma, beta)


# speedup vs baseline: 4.1124x; 1.0489x over previous
import functools

import jax
import jax.numpy as jnp
from jax import lax
from jax.experimental import pallas as pl
from jax.experimental.pallas import tpu as pltpu
from jax.experimental.pallas import tpu_sc as plsc

B = 4096
L = 200
H = 64
LH = L // 2
EPS = 1e-5
NC = 2
NS = 16
NW = NC * NS
ROWS_PER_W = B // NW
NG = (L + 15) // 16
LP = NG * 16
NBUF = 4


def _rsqrt(x):
    i = plsc.bitcast(x, jnp.int32)
    i = 0x5F3759DF - (i >> 1)
    y = plsc.bitcast(i, jnp.float32)
    for _ in range(3):
        y = y * (1.5 - 0.5 * x * y * y)
    return y


def kernel(input_ids, word_emb, pos_emb, gamma, beta):
    ids2 = input_ids.reshape(2 * B, LH).astype(jnp.int32)
    mesh = plsc.VectorSubcoreMesh(core_axis_name="c", subcore_axis_name="s")

    @functools.partial(
        pl.kernel,
        out_type=jax.ShapeDtypeStruct((B, L, H), jnp.float32),
        mesh=mesh,
        compiler_params=pltpu.CompilerParams(
            needs_layout_passes=False, use_tc_tiling_on_sc=False),
        scratch_types=[
            pltpu.VMEM((2 * ROWS_PER_W, LH), jnp.int32),
            pltpu.VMEM((NBUF, L, H), jnp.float32),
            pltpu.VMEM((H, LP), jnp.float32),
            pltpu.VMEM((H,), jnp.float32),
            pltpu.VMEM((H,), jnp.float32),
            pltpu.VMEM((H, 16), jnp.float32),
            pltpu.VMEM((H, 16), jnp.float32),
            pltpu.VMEM((H, 16), jnp.float32),
            pltpu.VMEM((2, L, H), jnp.float32),
            pltpu.SemaphoreType.DMA,
            pltpu.SemaphoreType.DMA,
            pltpu.SemaphoreType.DMA,
            pltpu.SemaphoreType.DMA,
            pltpu.SemaphoreType.DMA,
            pltpu.SemaphoreType.DMA,
        ],
    )
    def run(ids_hbm, wemb_hbm, pemb_hbm, gamma_hbm, beta_hbm, out_hbm,
            idx_v, rows_v, pos_t, g_v, b_v, x_t, g_rot, b_rot, out_v,
            gsem0, gsem1, gsem2, gsem3, osem0, osem1):
        gsems = [gsem0, gsem1, gsem2, gsem3]
        osems = [osem0, osem1]
        wid = lax.axis_index("s") * NC + lax.axis_index("c")
        row0 = wid * ROWS_PER_W
        pltpu.sync_copy(ids_hbm.at[pl.ds(2 * row0, 2 * ROWS_PER_W)], idx_v)
        pltpu.sync_copy(gamma_hbm, g_v)
        pltpu.sync_copy(beta_hbm, b_v)
        iota = lax.iota(jnp.int32, 16)

        def ecol(h):
            return (iota + h) & (H - 1)

        def build_rot(h, carry):
            g_rot[h, pl.ds(0, 16)] = plsc.load_gather(g_v, [ecol(h)])
            b_rot[h, pl.ds(0, 16)] = plsc.load_gather(b_v, [ecol(h)])
            return carry

        lax.fori_loop(0, H, build_rot, 0)

        pltpu.sync_copy(pemb_hbm.at[pl.ds(0, L)], rows_v.at[0])

        def transpose_pos(g, carry):
            tok = jnp.minimum(g * 16 + iota, L - 1)

            def tp_h(h, c):
                pos_t[h, pl.ds(g * 16, 16)] = plsc.load_gather(
                    rows_v.at[0], [tok, ecol(h)])
                return c

            lax.fori_loop(0, H, tp_h, 0)
            return carry

        lax.fori_loop(0, NG, transpose_pos, 0)

        def issue_gather(r, buf):
            pltpu.async_copy(
                wemb_hbm.at[idx_v.at[2 * r]],
                rows_v.at[buf, pl.ds(0, LH)], gsems[buf])
            pltpu.async_copy(
                wemb_hbm.at[idx_v.at[2 * r + 1]],
                rows_v.at[buf, pl.ds(LH, LH)], gsems[buf])

        def wait_gather(r, buf):
            for j in range(2):
                pltpu.make_async_copy(
                    wemb_hbm.at[idx_v.at[2 * r + j]],
                    rows_v.at[buf, pl.ds(j * LH, LH)], gsems[buf]).wait()

        def compute_row(r, buf, obuf):
            def grp_body(g, carry):
                tok = jnp.minimum(g * 16 + iota, L - 1)
                zero = jnp.zeros((16,), jnp.float32)

                @plsc.parallel_loop(0, H, unroll=8,
                                    carry=(zero, zero, zero, zero))
                def pass1(h, c):
                    s0, q0, s1, q1 = c
                    w = plsc.load_gather(rows_v.at[buf], [tok, ecol(h)])
                    x = w + pos_t[h, pl.ds(g * 16, 16)]
                    x_t[h, pl.ds(0, 16)] = x
                    return (s1, q1, s0 + x, q0 + x * x)

                s0, q0, s1, q1 = pass1
                mean = (s0 + s1) * (1.0 / H)
                var = (q0 + q1) * (1.0 / H) - mean * mean
                inv = _rsqrt(var + EPS)

                @plsc.parallel_loop(0, H, unroll=16)
                def pass2(h):
                    x = x_t[h, pl.ds(0, 16)]
                    y = (x - mean) * inv
                    y = y * g_rot[h, pl.ds(0, 16)] + b_rot[h, pl.ds(0, 16)]
                    plsc.store_scatter(out_v.at[obuf], [tok, ecol(h)], y)
                return carry

            lax.fori_loop(0, NG, grp_body, 0)

        def wait_out(r, obuf):
            pltpu.make_async_copy(
                out_v.at[obuf], out_hbm.at[row0 + r], osems[obuf]).wait()

        def issue_out(r, obuf):
            pltpu.async_copy(out_v.at[obuf], out_hbm.at[row0 + r], osems[obuf])

        for b in range(NBUF - 1):
            issue_gather(b, b)

        def quad_body(i, carry):
            for b in range(NBUF):
                r = NBUF * i + b
                nb = (b + NBUF - 1) % NBUF

                @pl.when(r + NBUF - 1 < ROWS_PER_W)
                def _():
                    issue_gather(r + NBUF - 1, nb)
                wait_gather(r, b)
                ob = b % 2

                @pl.when(r > 1)
                def _():
                    wait_out(r - 2, ob)
                compute_row(r, b, ob)
                issue_out(r, ob)
            return carry

        lax.fori_loop(0, ROWS_PER_W // NBUF, quad_body, 0)
        wait_out(ROWS_PER_W - 2, 0)
        wait_out(ROWS_PER_W - 1, 1)

    return run(ids2, word_emb, pos_emb, gamma, beta)
